# baseline (speedup 1.0000x reference)
import functools

import jax
import jax.numpy as jnp
from jax import lax
from jax.experimental import pallas as pl
from jax.experimental.pallas import tpu as pltpu
from jax.experimental.pallas import tpu_sc as plsc

D = 32
L = 16
NC = 2
NS = 16
NW = NC * NS
BCH = 128


def _fire_chunk(idx_v, table_hbm, rows_v, b0, sem0, sem1):
    def go(f, carry):
        r0 = pl.multiple_of(f * L, L)
        vec = idx_v[pl.ds(b0 + r0, L)]
        for u in range(L):
            pltpu.async_copy(table_hbm.at[pl.ds(vec[u], 1)],
                             rows_v.at[pl.ds(r0 + u, 1)],
                             sem0 if u % 2 == 0 else sem1)
        return carry
    lax.fori_loop(0, BCH // L, go, 0)


def _drain_chunk(table_hbm, rows_v, sem0, sem1):
    pltpu.make_async_copy(table_hbm.at[pl.ds(0, BCH // 2)],
                          rows_v.at[pl.ds(0, BCH // 2)], sem0).wait()
    pltpu.make_async_copy(table_hbm.at[pl.ds(0, BCH // 2)],
                          rows_v.at[pl.ds(0, BCH // 2)], sem1).wait()


def _sci_body(b_per_w, sid_hbm, semb_hbm, srows_hbm,
              sidx_v, rows_v, sem0, sem1):
    wid = lax.axis_index("s") * NC + lax.axis_index("c")
    base = pl.multiple_of(wid * b_per_w, b_per_w)
    pltpu.sync_copy(sid_hbm.at[pl.ds(base, b_per_w)], sidx_v)

    def chunk(c, carry):
        b0 = pl.multiple_of(c * BCH, BCH)
        _fire_chunk(sidx_v, semb_hbm, rows_v, b0, sem0, sem1)
        _drain_chunk(semb_hbm, rows_v, sem0, sem1)
        pltpu.sync_copy(rows_v, srows_hbm.at[pl.ds(base + b0, BCH)])
        return carry

    lax.fori_loop(0, b_per_w // BCH, chunk, 0)


def _dot_body(b_per_w, pid_hbm, pemb_hbm, srows_hbm, out_hbm,
              pidx_v, srows_v, prows_v, out_v, sem0, sem1):
    wid = lax.axis_index("s") * NC + lax.axis_index("c")
    base = pl.multiple_of(wid * b_per_w, b_per_w)

    pltpu.sync_copy(pid_hbm.at[pl.ds(base, b_per_w)], pidx_v)
    pltpu.sync_copy(srows_hbm.at[pl.ds(base, b_per_w)], srows_v)

    lane = lax.iota(jnp.int32, L)
    last_lane = lane == (L - 1)

    def chunk(c, carry):
        b0 = pl.multiple_of(c * BCH, BCH)
        _fire_chunk(pidx_v, pemb_hbm, prows_v, b0, sem0, sem1)
        _drain_chunk(pemb_hbm, prows_v, sem0, sem1)

        def compute(g, carry2):
            r0 = pl.multiple_of(g * L, L)
            for u in range(L):
                q = (srows_v[b0 + r0 + u, pl.ds(0, L)]
                     * prows_v[r0 + u, pl.ds(0, L)]
                     + srows_v[b0 + r0 + u, pl.ds(L, L)]
                     * prows_v[r0 + u, pl.ds(L, L)])
                cum = plsc.cumsum(q)
                plsc.store_scatter(
                    out_v, [jnp.full((L,), b0 + r0 + u, jnp.int32)],
                    cum, mask=last_lane)
            return carry2

        lax.fori_loop(0, BCH // L, compute, 0)
        return carry

    lax.fori_loop(0, b_per_w // BCH, chunk, 0)
    pltpu.sync_copy(out_v, out_hbm.at[pl.ds(base, b_per_w)])


def kernel(sid, pid, scientist_emb, paper_emb):
    batch = sid.shape[0]
    b_per_w = batch // NW
    mesh = plsc.VectorSubcoreMesh(core_axis_name="c", subcore_axis_name="s",
                                  num_cores=NC, num_subcores=NS)
    params = pltpu.CompilerParams(needs_layout_passes=False,
                                  use_tc_tiling_on_sc=True)

    gather_sci = pl.kernel(
        functools.partial(_sci_body, b_per_w),
        out_type=jax.ShapeDtypeStruct((batch, D), jnp.float32),
        mesh=mesh,
        scratch_types=[
            pltpu.VMEM((b_per_w,), jnp.int32),
            pltpu.VMEM((BCH, D), jnp.float32),
            pltpu.SemaphoreType.DMA,
            pltpu.SemaphoreType.DMA,
        ],
        compiler_params=params,
    )
    s_rows = gather_sci(sid.astype(jnp.int32), scientist_emb)

    dot = pl.kernel(
        functools.partial(_dot_body, b_per_w),
        out_type=jax.ShapeDtypeStruct((batch,), jnp.float32),
        mesh=mesh,
        scratch_types=[
            pltpu.VMEM((b_per_w,), jnp.int32),
            pltpu.VMEM((b_per_w, D), jnp.float32),
            pltpu.VMEM((BCH, D), jnp.float32),
            pltpu.VMEM((b_per_w,), jnp.float32),
            pltpu.SemaphoreType.DMA,
            pltpu.SemaphoreType.DMA,
        ],
        compiler_params=params,
    )
    return dot(pid.astype(jnp.int32), paper_emb, s_rows)


# speedup vs baseline: 1.5742x; 1.5742x over previous
import functools

import jax
import jax.numpy as jnp
from jax import lax
from jax.experimental import pallas as pl
from jax.experimental.pallas import tpu as pltpu
from jax.experimental.pallas import tpu_sc as plsc

D = 32
L = 16
NC = 2
NS = 16
NW = NC * NS
BCH = 128


def _fire_chunk(idx_v, table_hbm, rows_v, b0, sem0, sem1):
    def go(f, carry):
        r0 = pl.multiple_of(f * L, L)
        vec = idx_v[pl.ds(b0 + r0, L)]
        for u in range(L):
            t = vec[u] >> 3
            r = vec[u] & 7
            pltpu.async_copy(table_hbm.at[t, pl.ds(r, 1)],
                             rows_v.at[pl.ds(r0 + u, 1)],
                             sem0 if u % 2 == 0 else sem1)
        return carry
    lax.fori_loop(0, BCH // L, go, 0)


def _drain_chunk(dummy_hbm, rows_v, sem0, sem1):
    pltpu.make_async_copy(dummy_hbm.at[pl.ds(0, BCH // 2)],
                          rows_v.at[pl.ds(0, BCH // 2)], sem0).wait()
    pltpu.make_async_copy(dummy_hbm.at[pl.ds(0, BCH // 2)],
                          rows_v.at[pl.ds(0, BCH // 2)], sem1).wait()


def _paper_body(b_per_w, pid_hbm, pemb_hbm, prows_hbm,
                pidx_v, rows_v, sem0, sem1):
    wid = lax.axis_index("s") * NC + lax.axis_index("c")
    base = pl.multiple_of(wid * b_per_w, b_per_w)
    pltpu.sync_copy(pid_hbm.at[pl.ds(base, b_per_w)], pidx_v)

    def chunk(c, carry):
        b0 = pl.multiple_of(c * BCH, BCH)
        _fire_chunk(pidx_v, pemb_hbm, rows_v, b0, sem0, sem1)
        _drain_chunk(prows_hbm, rows_v, sem0, sem1)
        pltpu.sync_copy(rows_v, prows_hbm.at[pl.ds(base + b0, BCH)])
        return carry

    lax.fori_loop(0, b_per_w // BCH, chunk, 0)


def _dot_body(b_per_w, sid_hbm, semb_hbm, prows_hbm, out_hbm,
              sidx_v, prows_v, srows_v, out_v, sem0, sem1):
    wid = lax.axis_index("s") * NC + lax.axis_index("c")
    base = pl.multiple_of(wid * b_per_w, b_per_w)

    pltpu.sync_copy(sid_hbm.at[pl.ds(base, b_per_w)], sidx_v)
    pltpu.sync_copy(prows_hbm.at[pl.ds(base, b_per_w)], prows_v)

    lane = lax.iota(jnp.int32, L)
    last_lane = lane == (L - 1)

    def chunk(c, carry):
        b0 = pl.multiple_of(c * BCH, BCH)
        _fire_chunk(sidx_v, semb_hbm, srows_v, b0, sem0, sem1)
        _drain_chunk(prows_hbm, srows_v, sem0, sem1)

        def compute(g, carry2):
            r0 = pl.multiple_of(g * L, L)
            for u in range(L):
                q = (srows_v[r0 + u, pl.ds(0, L)]
                     * prows_v[b0 + r0 + u, pl.ds(0, L)]
                     + srows_v[r0 + u, pl.ds(L, L)]
                     * prows_v[b0 + r0 + u, pl.ds(L, L)])
                cum = plsc.cumsum(q)
                plsc.store_scatter(
                    out_v, [jnp.full((L,), b0 + r0 + u, jnp.int32)],
                    cum, mask=last_lane)
            return carry2

        lax.fori_loop(0, BCH // L, compute, 0)
        return carry

    lax.fori_loop(0, b_per_w // BCH, chunk, 0)
    pltpu.sync_copy(out_v, out_hbm.at[pl.ds(base, b_per_w)])


def kernel(sid, pid, scientist_emb, paper_emb):
    batch = sid.shape[0]
    b_per_w = batch // NW
    mesh = plsc.VectorSubcoreMesh(core_axis_name="c", subcore_axis_name="s",
                                  num_cores=NC, num_subcores=NS)
    params = pltpu.CompilerParams(needs_layout_passes=False,
                                  use_tc_tiling_on_sc=True)

    gather_paper = pl.kernel(
        functools.partial(_paper_body, b_per_w),
        out_type=jax.ShapeDtypeStruct((batch, D), jnp.float32),
        mesh=mesh,
        scratch_types=[
            pltpu.VMEM((b_per_w,), jnp.int32),
            pltpu.VMEM((BCH, D), jnp.float32),
            pltpu.SemaphoreType.DMA,
            pltpu.SemaphoreType.DMA,
        ],
        compiler_params=params,
    )
    p_rows = gather_paper(pid.astype(jnp.int32),
                          paper_emb.reshape(-1, 8, D))

    dot = pl.kernel(
        functools.partial(_dot_body, b_per_w),
        out_type=jax.ShapeDtypeStruct((batch,), jnp.float32),
        mesh=mesh,
        scratch_types=[
            pltpu.VMEM((b_per_w,), jnp.int32),
            pltpu.VMEM((b_per_w, D), jnp.float32),
            pltpu.VMEM((BCH, D), jnp.float32),
            pltpu.VMEM((b_per_w,), jnp.float32),
            pltpu.SemaphoreType.DMA,
            pltpu.SemaphoreType.DMA,
        ],
        compiler_params=params,
    )
    return dot(sid.astype(jnp.int32), scientist_emb.reshape(-1, 8, D),
               p_rows)


# speedup vs baseline: 1.6702x; 1.0610x over previous
import functools

import jax
import jax.numpy as jnp
from jax import lax
from jax.experimental import pallas as pl
from jax.experimental.pallas import tpu as pltpu
from jax.experimental.pallas import tpu_sc as plsc

D = 32
TR = 8
L = 16
NC = 2
NS = 16
NW = NC * NS
BCH = 128


def _fire_chunk(idx_v, table_hbm, rows_v, i0, d0, sem0, sem1):
    def go(f, carry):
        r0 = pl.multiple_of(f * L, L)
        vec = idx_v[pl.ds(i0 + r0, L)]
        for u in range(L):
            t = vec[u] >> 3
            r = vec[u] & 7
            pltpu.async_copy(table_hbm.at[t, pl.ds(r, 1)],
                             rows_v.at[pl.ds(d0 + r0 + u, 1)],
                             sem0 if u % 2 == 0 else sem1)
        return carry
    lax.fori_loop(0, BCH // L, go, 0)


def _drain_chunk(table_hbm, rows_v, sem0, sem1):
    for _ in range(BCH // 2 // TR):
        pltpu.make_async_copy(table_hbm.at[0],
                              rows_v.at[pl.ds(0, TR)], sem0).wait()
        pltpu.make_async_copy(table_hbm.at[0],
                              rows_v.at[pl.ds(0, TR)], sem1).wait()


def _dot_body(b_per_w, sid_hbm, pid_hbm, semb_hbm, pemb_hbm, out_hbm,
              sidx_v, pidx_v, srows_v, prows_v, out_v, *sems):
    wid = lax.axis_index("s") * NC + lax.axis_index("c")
    base = pl.multiple_of(wid * b_per_w, b_per_w)

    pltpu.sync_copy(sid_hbm.at[pl.ds(base, b_per_w)], sidx_v)
    pltpu.sync_copy(pid_hbm.at[pl.ds(base, b_per_w)], pidx_v)

    lane = lax.iota(jnp.int32, L)
    last_lane = lane == (L - 1)

    def pchunk(c, carry):
        b0 = pl.multiple_of(c * BCH, BCH)
        _fire_chunk(pidx_v, pemb_hbm, prows_v, b0, b0, sems[0], sems[1])
        return carry

    lax.fori_loop(0, b_per_w // BCH, pchunk, 0)

    def schunk(c, carry):
        b0 = pl.multiple_of(c * BCH, BCH)
        _fire_chunk(sidx_v, semb_hbm, srows_v, b0, 0, sems[2], sems[3])
        _drain_chunk(semb_hbm, srows_v, sems[2], sems[3])

        @pl.when(c == 0)
        def _():
            for _k in range(b_per_w // BCH):
                _drain_chunk(pemb_hbm, prows_v, sems[0], sems[1])

        def compute(g, carry2):
            r0 = pl.multiple_of(g * L, L)
            for u in range(L):
                q = (srows_v[r0 + u, pl.ds(0, L)]
                     * prows_v[b0 + r0 + u, pl.ds(0, L)]
                     + srows_v[r0 + u, pl.ds(L, L)]
                     * prows_v[b0 + r0 + u, pl.ds(L, L)])
                cum = plsc.cumsum(q)
                plsc.store_scatter(
                    out_v, [jnp.full((L,), b0 + r0 + u, jnp.int32)],
                    cum, mask=last_lane)
            return carry2

        lax.fori_loop(0, BCH // L, compute, 0)
        return carry

    lax.fori_loop(0, b_per_w // BCH, schunk, 0)
    pltpu.sync_copy(out_v, out_hbm.at[pl.ds(base, b_per_w)])


def kernel(sid, pid, scientist_emb, paper_emb):
    batch = sid.shape[0]
    b_per_w = batch // NW
    mesh = plsc.VectorSubcoreMesh(core_axis_name="c", subcore_axis_name="s",
                                  num_cores=NC, num_subcores=NS)
    k = pl.kernel(
        functools.partial(_dot_body, b_per_w),
        out_type=jax.ShapeDtypeStruct((batch,), jnp.float32),
        mesh=mesh,
        scratch_types=[
            pltpu.VMEM((b_per_w,), jnp.int32),
            pltpu.VMEM((b_per_w,), jnp.int32),
            pltpu.VMEM((BCH, D), jnp.float32),
            pltpu.VMEM((b_per_w, D), jnp.float32),
            pltpu.VMEM((b_per_w,), jnp.float32),
        ] + [pltpu.SemaphoreType.DMA] * 4,
        compiler_params=pltpu.CompilerParams(needs_layout_passes=False,
                                             use_tc_tiling_on_sc=True),
    )
    return k(sid.astype(jnp.int32), pid.astype(jnp.int32),
             scientist_emb.reshape(-1, TR, D), paper_emb.reshape(-1, TR, D))
